# baseline (speedup 1.0000x reference)
import functools

import jax
import jax.numpy as jnp
import numpy as np
from jax import lax
from jax.experimental import pallas as pl
from jax.experimental.pallas import tpu as pltpu
from jax.experimental.pallas import tpu_sc as plsc

V = 10000
F = 10000
E = 160000
D = 256
H = 128
NS = 16
CH = 80
EPT = E // NS
BLK = 2000
NBLK = EPT // BLK
CPB = BLK // CH
RB = 1000
NB = F // RB

_f32 = jnp.float32


def _pre_body(hp, ho, w1, b1, w3, b3, w2, b2, w4, b4,
              td1, td2, c1, ts1, ts2, c2):
    hp_x = hp[:]
    ho_x = ho[:]
    y = jnp.dot(ho_x, w1[0:D], preferred_element_type=_f32) + b1[:]
    td1[0] = y[:, :H]
    td1[1] = y[:, H:]
    y = jnp.dot(ho_x, w3[D:2 * D], preferred_element_type=_f32)
    td2[0] = y[:, :H]
    td2[1] = y[:, H:]
    c1[:] = jnp.dot(ho_x, w2[0:D], preferred_element_type=_f32) + b2[:]
    y = jnp.dot(hp_x, w1[D:2 * D], preferred_element_type=_f32)
    ts1[0] = y[:, :H]
    ts1[1] = y[:, H:]
    y = jnp.dot(hp_x, w3[0:D], preferred_element_type=_f32) + b3[:]
    ts2[0] = y[:, :H]
    ts2[1] = y[:, H:]
    c2[:] = jnp.dot(hp_x, w4[0:D], preferred_element_type=_f32) + b4[:]


def _precompute(h_p, h_o, w1, b1, w3, b3, w2, b2, w4, b4, interpret=False):
    whole_w = pl.BlockSpec((2 * D, D), lambda i: (0, 0))
    whole_b = pl.BlockSpec((1, D), lambda i: (0, 0))
    row = pl.BlockSpec((RB, D), lambda i: (i, 0))
    half = pl.BlockSpec((2, RB, H), lambda i: (0, i, 0))
    return pl.pallas_call(
        _pre_body,
        grid=(NB,),
        in_specs=[row, row, whole_w, whole_b, whole_w, whole_b,
                  whole_w, whole_b, whole_w, whole_b],
        out_specs=[half, half, row, half, half, row],
        out_shape=[
            jax.ShapeDtypeStruct((2, F, H), _f32),
            jax.ShapeDtypeStruct((2, F, H), _f32),
            jax.ShapeDtypeStruct((F, D), _f32),
            jax.ShapeDtypeStruct((2, V, H), _f32),
            jax.ShapeDtypeStruct((2, V, H), _f32),
            jax.ShapeDtypeStruct((V, D), _f32),
        ],
        interpret=interpret,
    )(h_p, h_o, w1, b1, w3, b3, w2, b2, w4, b4)


ZB = F // CH
ZBT = -(-ZB // NS)


def _sc_body(td1, ts1, ts2, td2, src1, dst1, aggF, aggV,
             ids, idd, igs, igd, scx0, scx1,
             bufA0, bufB0, bufA1, bufB1,
             sA0, sB0, sA1, sB1, acc):
    cid = lax.axis_index("c")
    sid = lax.axis_index("s")
    e0 = sid * EPT

    off = cid * F

    def phase(tbl_a, tbl_b, a_by_dst, out):
        @pl.loop(0, CH)
        def _zb(r):
            for j in range(H // 16):
                bufA0[r, pl.ds(j * 16, 16)] = jnp.zeros((16,), _f32)

        @pl.loop(0, ZBT)
        def _z(t):
            b = sid + NS * t

            @pl.when(b < ZB)
            def _():
                pltpu.sync_copy(bufA0, acc.at[pl.ds(b * CH, CH)])

        plsc.subcore_barrier()

        ig_a, ig_b = (igd, igs) if a_by_dst else (igs, igd)
        id_main = idd if a_by_dst else ids

        def g_issue(k, ba, bb, sa, sb):
            c0 = k * CH
            pltpu.async_copy(tbl_a.at[ig_a.at[pl.ds(c0, CH)]], ba, sa)
            pltpu.async_copy(tbl_b.at[ig_b.at[pl.ds(c0, CH)]], bb, sb)

        def g_wait(ba, bb, sa, sb):
            pltpu.make_async_copy(
                tbl_a.at[ig_a.at[pl.ds(0, CH)]], ba, sa).wait()
            pltpu.make_async_copy(
                tbl_b.at[ig_b.at[pl.ds(0, CH)]], bb, sb).wait()

        def proc(k, ba, bb, sx):
            for j in range(CH // 16):
                sx[pl.ds(j * 16, 16)] = id_main[pl.ds(k * CH + j * 16, 16)]

            @pl.loop(0, CH)
            def _relu(r):
                for j in range(H // 16):
                    s = pl.ds(j * 16, 16)
                    ba[r, s] = jnp.maximum(ba[r, s] + bb[r, s], 0.0)

            pltpu.sync_copy(ba, acc.at[sx], add=True)

        @pl.loop(0, NBLK)
        def _blk(bi):
            base = e0 + bi * BLK
            pltpu.sync_copy(src1.at[pl.ds(base, BLK)], ids)
            pltpu.sync_copy(dst1.at[pl.ds(base, BLK)], idd)

            @pl.loop(0, BLK // 16)
            def _offs(i):
                s = pl.ds(i * 16, 16)
                igs[s] = ids[s] + off
                igd[s] = idd[s] + off

            g_issue(0, bufA0, bufB0, sA0, sB0)

            @pl.loop(0, (CPB - 1) // 2)
            def _pair(kk):
                k0 = kk * 2
                g_issue(k0 + 1, bufA1, bufB1, sA1, sB1)
                g_wait(bufA0, bufB0, sA0, sB0)
                proc(k0, bufA0, bufB0, scx0)
                g_issue(k0 + 2, bufA0, bufB0, sA0, sB0)
                g_wait(bufA1, bufB1, sA1, sB1)
                proc(k0 + 1, bufA1, bufB1, scx1)

            g_wait(bufA0, bufB0, sA0, sB0)
            proc(CPB - 1, bufA0, bufB0, scx0)

        plsc.subcore_barrier()

        @pl.loop(0, ZBT)
        def _w(t):
            b = sid + NS * t

            @pl.when(b < ZB)
            def _():
                pltpu.sync_copy(acc.at[pl.ds(b * CH, CH)],
                                out.at[cid, pl.ds(b * CH, CH)])

        plsc.subcore_barrier()

    phase(td1, ts1, True, aggF)
    phase(ts2, td2, False, aggV)


def _edge_sc(td1, ts1, ts2, td2, src1, dst1):
    mesh = plsc.VectorSubcoreMesh(core_axis_name="c", subcore_axis_name="s")
    fn = pl.kernel(
        _sc_body,
        out_type=[
            jax.ShapeDtypeStruct((2, F, H), _f32),
            jax.ShapeDtypeStruct((2, V, H), _f32),
        ],
        mesh=mesh,
        scratch_types=[
            pltpu.VMEM((BLK,), jnp.int32),
            pltpu.VMEM((BLK,), jnp.int32),
            pltpu.VMEM((BLK,), jnp.int32),
            pltpu.VMEM((BLK,), jnp.int32),
            pltpu.VMEM((CH,), jnp.int32),
            pltpu.VMEM((CH,), jnp.int32),
            pltpu.VMEM((CH, H), _f32),
            pltpu.VMEM((CH, H), _f32),
            pltpu.VMEM((CH, H), _f32),
            pltpu.VMEM((CH, H), _f32),
            pltpu.SemaphoreType.DMA,
            pltpu.SemaphoreType.DMA,
            pltpu.SemaphoreType.DMA,
            pltpu.SemaphoreType.DMA,
            pltpu.VMEM_SHARED((F, H), _f32),
        ],
    )
    return fn(td1, ts1, ts2, td2, src1, dst1)


def _comb_body(hp, af, av, c1, c2, w2, w4, out_o, out_p):
    acc_o = (c1[:]
             + jnp.dot(af[0], w2[D:D + H], preferred_element_type=_f32)
             + jnp.dot(af[1], w2[D + H:2 * D], preferred_element_type=_f32))
    out_o[:] = jnp.maximum(acc_o, 0.0)
    acc_p = (c2[:]
             + jnp.dot(av[0], w4[D:D + H], preferred_element_type=_f32)
             + jnp.dot(av[1], w4[D + H:2 * D], preferred_element_type=_f32))
    out_p[:] = hp[:] + jnp.maximum(acc_p, 0.0)


def _combine(h_p, aggF, aggV, c1, c2, w2, w4, interpret=False):
    whole_w = pl.BlockSpec((2 * D, D), lambda i: (0, 0))
    row = pl.BlockSpec((RB, D), lambda i: (i, 0))
    half = pl.BlockSpec((2, RB, H), lambda i: (0, i, 0))
    return pl.pallas_call(
        _comb_body,
        grid=(NB,),
        in_specs=[row, half, half, row, row, whole_w, whole_w],
        out_specs=[row, row],
        out_shape=[
            jax.ShapeDtypeStruct((F, D), _f32),
            jax.ShapeDtypeStruct((V, D), _f32),
        ],
        interpret=interpret,
    )(h_p, aggF, aggV, c1, c2, w2, w4)


def kernel(h_p, h_o, edge_index, edge_attr,
           W_v2f_msg, b_v2f_msg, W_v2f_comb, b_v2f_comb,
           W_f2v_msg, b_f2v_msg, W_f2v_comb, b_f2v_comb):
    src = edge_index[0].astype(jnp.int32)
    dst = edge_index[1].astype(jnp.int32)
    b1 = b_v2f_msg.reshape(1, D)
    b2 = b_v2f_comb.reshape(1, D)
    b3 = b_f2v_msg.reshape(1, D)
    b4 = b_f2v_comb.reshape(1, D)

    td1, td2, c1, ts1, ts2, c2 = _precompute(
        h_p, h_o, W_v2f_msg, b1, W_f2v_msg, b3, W_v2f_comb, b2, W_f2v_comb, b4)

    aggF, aggV = _edge_sc(
        td1.reshape(2 * F, H), ts1.reshape(2 * V, H),
        ts2.reshape(2 * V, H), td2.reshape(2 * F, H), src, dst)

    n_h_o, n_h_p = _combine(h_p, aggF, aggV, c1, c2, W_v2f_comb, W_f2v_comb)
    return (n_h_p, n_h_o)


# speedup vs baseline: 1.0025x; 1.0025x over previous
import functools

import jax
import jax.numpy as jnp
import numpy as np
from jax import lax
from jax.experimental import pallas as pl
from jax.experimental.pallas import tpu as pltpu
from jax.experimental.pallas import tpu_sc as plsc

V = 10000
F = 10000
E = 160000
D = 256
H = 128
NS = 16
CH = 80
EPT = E // NS
BLK = 2000
NBLK = EPT // BLK
CPB = BLK // CH
RB = 1000
NB = F // RB

_f32 = jnp.float32


def _pre_body(hp, ho, w1, b1, w3, b3, w2, b2, w4, b4,
              td1, td2, c1, ts1, ts2, c2):
    hp_x = hp[:]
    ho_x = ho[:]
    y = jnp.dot(ho_x, w1[0:D], preferred_element_type=_f32) + b1[:]
    td1[0] = y[:, :H]
    td1[1] = y[:, H:]
    y = jnp.dot(ho_x, w3[D:2 * D], preferred_element_type=_f32)
    td2[0] = y[:, :H]
    td2[1] = y[:, H:]
    c1[:] = jnp.dot(ho_x, w2[0:D], preferred_element_type=_f32) + b2[:]
    y = jnp.dot(hp_x, w1[D:2 * D], preferred_element_type=_f32)
    ts1[0] = y[:, :H]
    ts1[1] = y[:, H:]
    y = jnp.dot(hp_x, w3[0:D], preferred_element_type=_f32) + b3[:]
    ts2[0] = y[:, :H]
    ts2[1] = y[:, H:]
    c2[:] = jnp.dot(hp_x, w4[0:D], preferred_element_type=_f32) + b4[:]


def _precompute(h_p, h_o, w1, b1, w3, b3, w2, b2, w4, b4, interpret=False):
    whole_w = pl.BlockSpec((2 * D, D), lambda i: (0, 0))
    whole_b = pl.BlockSpec((1, D), lambda i: (0, 0))
    row = pl.BlockSpec((RB, D), lambda i: (i, 0))
    half = pl.BlockSpec((2, RB, H), lambda i: (0, i, 0))
    return pl.pallas_call(
        _pre_body,
        grid=(NB,),
        in_specs=[row, row, whole_w, whole_b, whole_w, whole_b,
                  whole_w, whole_b, whole_w, whole_b],
        out_specs=[half, half, row, half, half, row],
        out_shape=[
            jax.ShapeDtypeStruct((2, F, H), _f32),
            jax.ShapeDtypeStruct((2, F, H), _f32),
            jax.ShapeDtypeStruct((F, D), _f32),
            jax.ShapeDtypeStruct((2, V, H), _f32),
            jax.ShapeDtypeStruct((2, V, H), _f32),
            jax.ShapeDtypeStruct((V, D), _f32),
        ],
        interpret=interpret,
    )(h_p, h_o, w1, b1, w3, b3, w2, b2, w4, b4)


ZB = F // CH
ZBT = -(-ZB // NS)


def _sc_body(td1, ts1, ts2, td2, src1, dst1, aggF, aggV,
             ids, idd, igs, igd, scx0, scx1,
             bufA0, bufB0, bufA1, bufB1,
             sA0, sB0, sA1, sB1, acc):
    cid = lax.axis_index("c")
    sid = lax.axis_index("s")
    e0 = sid * EPT

    off = cid * F

    def phase(tbl_a, tbl_b, a_by_dst, out):
        @pl.loop(0, CH // 4)
        def _zb(rq):
            r = rq * 4
            for rr in range(4):
                for j in range(H // 16):
                    bufA0[r + rr, pl.ds(j * 16, 16)] = jnp.zeros((16,), _f32)

        @pl.loop(0, ZBT)
        def _z(t):
            b = sid + NS * t

            @pl.when(b < ZB)
            def _():
                pltpu.sync_copy(bufA0, acc.at[pl.ds(b * CH, CH)])

        plsc.subcore_barrier()

        ig_a, ig_b = (igd, igs) if a_by_dst else (igs, igd)
        id_main = idd if a_by_dst else ids

        def g_issue(k, ba, bb, sa, sb):
            c0 = k * CH
            pltpu.async_copy(tbl_a.at[ig_a.at[pl.ds(c0, CH)]], ba, sa)
            pltpu.async_copy(tbl_b.at[ig_b.at[pl.ds(c0, CH)]], bb, sb)

        def g_wait(ba, bb, sa, sb):
            pltpu.make_async_copy(
                tbl_a.at[ig_a.at[pl.ds(0, CH)]], ba, sa).wait()
            pltpu.make_async_copy(
                tbl_b.at[ig_b.at[pl.ds(0, CH)]], bb, sb).wait()

        def proc(k, ba, bb, sx):
            for j in range(CH // 16):
                sx[pl.ds(j * 16, 16)] = id_main[pl.ds(k * CH + j * 16, 16)]

            @pl.loop(0, CH // 4)
            def _relu(rq):
                r = rq * 4
                for rr in range(4):
                    for j in range(H // 16):
                        s = pl.ds(j * 16, 16)
                        ba[r + rr, s] = jnp.maximum(ba[r + rr, s] + bb[r + rr, s],
                                                    0.0)

            pltpu.sync_copy(ba, acc.at[sx], add=True)

        @pl.loop(0, NBLK)
        def _blk(bi):
            base = e0 + bi * BLK
            pltpu.sync_copy(src1.at[pl.ds(base, BLK)], ids)
            pltpu.sync_copy(dst1.at[pl.ds(base, BLK)], idd)

            @pl.loop(0, BLK // 16)
            def _offs(i):
                s = pl.ds(i * 16, 16)
                igs[s] = ids[s] + off
                igd[s] = idd[s] + off

            g_issue(0, bufA0, bufB0, sA0, sB0)

            @pl.loop(0, (CPB - 1) // 2)
            def _pair(kk):
                k0 = kk * 2
                g_issue(k0 + 1, bufA1, bufB1, sA1, sB1)
                g_wait(bufA0, bufB0, sA0, sB0)
                proc(k0, bufA0, bufB0, scx0)
                g_issue(k0 + 2, bufA0, bufB0, sA0, sB0)
                g_wait(bufA1, bufB1, sA1, sB1)
                proc(k0 + 1, bufA1, bufB1, scx1)

            g_wait(bufA0, bufB0, sA0, sB0)
            proc(CPB - 1, bufA0, bufB0, scx0)

        plsc.subcore_barrier()

        @pl.loop(0, ZBT)
        def _w(t):
            b = sid + NS * t

            @pl.when(b < ZB)
            def _():
                pltpu.sync_copy(acc.at[pl.ds(b * CH, CH)],
                                out.at[cid, pl.ds(b * CH, CH)])

        plsc.subcore_barrier()

    phase(td1, ts1, True, aggF)
    phase(ts2, td2, False, aggV)


def _edge_sc(td1, ts1, ts2, td2, src1, dst1):
    mesh = plsc.VectorSubcoreMesh(core_axis_name="c", subcore_axis_name="s")
    fn = pl.kernel(
        _sc_body,
        out_type=[
            jax.ShapeDtypeStruct((2, F, H), _f32),
            jax.ShapeDtypeStruct((2, V, H), _f32),
        ],
        mesh=mesh,
        scratch_types=[
            pltpu.VMEM((BLK,), jnp.int32),
            pltpu.VMEM((BLK,), jnp.int32),
            pltpu.VMEM((BLK,), jnp.int32),
            pltpu.VMEM((BLK,), jnp.int32),
            pltpu.VMEM((CH,), jnp.int32),
            pltpu.VMEM((CH,), jnp.int32),
            pltpu.VMEM((CH, H), _f32),
            pltpu.VMEM((CH, H), _f32),
            pltpu.VMEM((CH, H), _f32),
            pltpu.VMEM((CH, H), _f32),
            pltpu.SemaphoreType.DMA,
            pltpu.SemaphoreType.DMA,
            pltpu.SemaphoreType.DMA,
            pltpu.SemaphoreType.DMA,
            pltpu.VMEM_SHARED((F, H), _f32),
        ],
    )
    return fn(td1, ts1, ts2, td2, src1, dst1)


def _comb_body(hp, af, av, c1, c2, w2, w4, out_o, out_p):
    acc_o = (c1[:]
             + jnp.dot(af[0], w2[D:D + H], preferred_element_type=_f32)
             + jnp.dot(af[1], w2[D + H:2 * D], preferred_element_type=_f32))
    out_o[:] = jnp.maximum(acc_o, 0.0)
    acc_p = (c2[:]
             + jnp.dot(av[0], w4[D:D + H], preferred_element_type=_f32)
             + jnp.dot(av[1], w4[D + H:2 * D], preferred_element_type=_f32))
    out_p[:] = hp[:] + jnp.maximum(acc_p, 0.0)


def _combine(h_p, aggF, aggV, c1, c2, w2, w4, interpret=False):
    whole_w = pl.BlockSpec((2 * D, D), lambda i: (0, 0))
    row = pl.BlockSpec((RB, D), lambda i: (i, 0))
    half = pl.BlockSpec((2, RB, H), lambda i: (0, i, 0))
    return pl.pallas_call(
        _comb_body,
        grid=(NB,),
        in_specs=[row, half, half, row, row, whole_w, whole_w],
        out_specs=[row, row],
        out_shape=[
            jax.ShapeDtypeStruct((F, D), _f32),
            jax.ShapeDtypeStruct((V, D), _f32),
        ],
        interpret=interpret,
    )(h_p, aggF, aggV, c1, c2, w2, w4)


def kernel(h_p, h_o, edge_index, edge_attr,
           W_v2f_msg, b_v2f_msg, W_v2f_comb, b_v2f_comb,
           W_f2v_msg, b_f2v_msg, W_f2v_comb, b_f2v_comb):
    src = edge_index[0].astype(jnp.int32)
    dst = edge_index[1].astype(jnp.int32)
    b1 = b_v2f_msg.reshape(1, D)
    b2 = b_v2f_comb.reshape(1, D)
    b3 = b_f2v_msg.reshape(1, D)
    b4 = b_f2v_comb.reshape(1, D)

    td1, td2, c1, ts1, ts2, c2 = _precompute(
        h_p, h_o, W_v2f_msg, b1, W_f2v_msg, b3, W_v2f_comb, b2, W_f2v_comb, b4)

    aggF, aggV = _edge_sc(
        td1.reshape(2 * F, H), ts1.reshape(2 * V, H),
        ts2.reshape(2 * V, H), td2.reshape(2 * F, H), src, dst)

    n_h_o, n_h_p = _combine(h_p, aggF, aggV, c1, c2, W_v2f_comb, W_f2v_comb)
    return (n_h_p, n_h_o)


# speedup vs baseline: 1.0093x; 1.0068x over previous
import functools

import jax
import jax.numpy as jnp
import numpy as np
from jax import lax
from jax.experimental import pallas as pl
from jax.experimental.pallas import tpu as pltpu
from jax.experimental.pallas import tpu_sc as plsc

V = 10000
F = 10000
E = 160000
D = 256
H = 128
NS = 16
CH = 80
EPT = E // NS
BLK = 2000
NBLK = EPT // BLK
CPB = BLK // CH
RB = 1000
NB = F // RB

_f32 = jnp.float32


def _pre_body(hp, ho, w1, b1, w3, b3, w2, b2, w4, b4,
              td1, td2, c1, ts1, ts2, c2):
    hp_x = hp[:]
    ho_x = ho[:]
    y = jnp.dot(ho_x, w1[0:D], preferred_element_type=_f32) + b1[:]
    td1[0] = y[:, :H]
    td1[1] = y[:, H:]
    y = jnp.dot(ho_x, w3[D:2 * D], preferred_element_type=_f32)
    td2[0] = y[:, :H]
    td2[1] = y[:, H:]
    c1[:] = jnp.dot(ho_x, w2[0:D], preferred_element_type=_f32) + b2[:]
    y = jnp.dot(hp_x, w1[D:2 * D], preferred_element_type=_f32)
    ts1[0] = y[:, :H]
    ts1[1] = y[:, H:]
    y = jnp.dot(hp_x, w3[0:D], preferred_element_type=_f32) + b3[:]
    ts2[0] = y[:, :H]
    ts2[1] = y[:, H:]
    c2[:] = jnp.dot(hp_x, w4[0:D], preferred_element_type=_f32) + b4[:]


def _precompute(h_p, h_o, w1, b1, w3, b3, w2, b2, w4, b4, interpret=False):
    whole_w = pl.BlockSpec((2 * D, D), lambda i: (0, 0))
    whole_b = pl.BlockSpec((1, D), lambda i: (0, 0))
    row = pl.BlockSpec((RB, D), lambda i: (i, 0))
    half = pl.BlockSpec((2, RB, H), lambda i: (0, i, 0))
    return pl.pallas_call(
        _pre_body,
        grid=(NB,),
        in_specs=[row, row, whole_w, whole_b, whole_w, whole_b,
                  whole_w, whole_b, whole_w, whole_b],
        out_specs=[half, half, row, half, half, row],
        out_shape=[
            jax.ShapeDtypeStruct((2, F, H), _f32),
            jax.ShapeDtypeStruct((2, F, H), _f32),
            jax.ShapeDtypeStruct((F, D), _f32),
            jax.ShapeDtypeStruct((2, V, H), _f32),
            jax.ShapeDtypeStruct((2, V, H), _f32),
            jax.ShapeDtypeStruct((V, D), _f32),
        ],
        interpret=interpret,
    )(h_p, h_o, w1, b1, w3, b3, w2, b2, w4, b4)


ZB = F // CH
ZBT = -(-ZB // NS)


def _sc_phase_body(a_by_dst, tbl_a, tbl_b, src1, dst1, out,
                   ids, idd, igs, igd, scx0, scx1,
                   bufA0, bufB0, bufA1, bufB1,
                   sA0, sB0, sA1, sB1, acc):
    cid = lax.axis_index("c")
    sid = lax.axis_index("s")
    e0 = sid * EPT

    off = cid * F

    @pl.loop(0, CH // 4)
    def _zb(rq):
        r = rq * 4
        for rr in range(4):
            for j in range(H // 16):
                bufA0[r + rr, pl.ds(j * 16, 16)] = jnp.zeros((16,), _f32)

    @pl.loop(0, ZBT)
    def _z(t):
        b = sid + NS * t

        @pl.when(b < ZB)
        def _():
            pltpu.sync_copy(bufA0, acc.at[pl.ds(b * CH, CH)])

    plsc.subcore_barrier()

    ig_a, ig_b = (igd, igs) if a_by_dst else (igs, igd)
    id_main = idd if a_by_dst else ids

    def g_issue(k, ba, bb, sa, sb):
        c0 = k * CH
        pltpu.async_copy(tbl_a.at[ig_a.at[pl.ds(c0, CH)]], ba, sa)
        pltpu.async_copy(tbl_b.at[ig_b.at[pl.ds(c0, CH)]], bb, sb)

    def g_wait(ba, bb, sa, sb):
        pltpu.make_async_copy(
            tbl_a.at[ig_a.at[pl.ds(0, CH)]], ba, sa).wait()
        pltpu.make_async_copy(
            tbl_b.at[ig_b.at[pl.ds(0, CH)]], bb, sb).wait()

    def proc(k, ba, bb, sx):
        for j in range(CH // 16):
            sx[pl.ds(j * 16, 16)] = id_main[pl.ds(k * CH + j * 16, 16)]

        @pl.loop(0, CH // 4)
        def _relu(rq):
            r = rq * 4
            for rr in range(4):
                for j in range(H // 16):
                    s = pl.ds(j * 16, 16)
                    ba[r + rr, s] = jnp.maximum(ba[r + rr, s] + bb[r + rr, s],
                                                0.0)

        pltpu.sync_copy(ba, acc.at[sx], add=True)

    @pl.loop(0, NBLK)
    def _blk(bi):
        base = e0 + bi * BLK
        pltpu.sync_copy(src1.at[pl.ds(base, BLK)], ids)
        pltpu.sync_copy(dst1.at[pl.ds(base, BLK)], idd)

        @pl.loop(0, BLK // 16)
        def _offs(i):
            s = pl.ds(i * 16, 16)
            igs[s] = ids[s] + off
            igd[s] = idd[s] + off

        g_issue(0, bufA0, bufB0, sA0, sB0)

        @pl.loop(0, (CPB - 1) // 2)
        def _pair(kk):
            k0 = kk * 2
            g_issue(k0 + 1, bufA1, bufB1, sA1, sB1)
            g_wait(bufA0, bufB0, sA0, sB0)
            proc(k0, bufA0, bufB0, scx0)
            g_issue(k0 + 2, bufA0, bufB0, sA0, sB0)
            g_wait(bufA1, bufB1, sA1, sB1)
            proc(k0 + 1, bufA1, bufB1, scx1)

        g_wait(bufA0, bufB0, sA0, sB0)
        proc(CPB - 1, bufA0, bufB0, scx0)

    plsc.subcore_barrier()

    @pl.loop(0, ZBT)
    def _w(t):
        b = sid + NS * t

        @pl.when(b < ZB)
        def _():
            pltpu.sync_copy(acc.at[pl.ds(b * CH, CH)],
                            out.at[cid, pl.ds(b * CH, CH)])


def _edge_sc_phase(tbl_a, tbl_b, src1, dst1, a_by_dst):
    mesh = plsc.VectorSubcoreMesh(core_axis_name="c", subcore_axis_name="s")
    fn = pl.kernel(
        functools.partial(_sc_phase_body, a_by_dst),
        out_type=jax.ShapeDtypeStruct((2, F, H), _f32),
        mesh=mesh,
        scratch_types=[
            pltpu.VMEM((BLK,), jnp.int32),
            pltpu.VMEM((BLK,), jnp.int32),
            pltpu.VMEM((BLK,), jnp.int32),
            pltpu.VMEM((BLK,), jnp.int32),
            pltpu.VMEM((CH,), jnp.int32),
            pltpu.VMEM((CH,), jnp.int32),
            pltpu.VMEM((CH, H), _f32),
            pltpu.VMEM((CH, H), _f32),
            pltpu.VMEM((CH, H), _f32),
            pltpu.VMEM((CH, H), _f32),
            pltpu.SemaphoreType.DMA,
            pltpu.SemaphoreType.DMA,
            pltpu.SemaphoreType.DMA,
            pltpu.SemaphoreType.DMA,
            pltpu.VMEM_SHARED((F, H), _f32),
        ],
    )
    return fn(tbl_a, tbl_b, src1, dst1)


def _comb_o_body(af, c1, w2, out_o):
    acc_o = (c1[:]
             + jnp.dot(af[0], w2[D:D + H], preferred_element_type=_f32)
             + jnp.dot(af[1], w2[D + H:2 * D], preferred_element_type=_f32))
    out_o[:] = jnp.maximum(acc_o, 0.0)


def _comb_p_body(hp, av, c2, w4, out_p):
    acc_p = (c2[:]
             + jnp.dot(av[0], w4[D:D + H], preferred_element_type=_f32)
             + jnp.dot(av[1], w4[D + H:2 * D], preferred_element_type=_f32))
    out_p[:] = hp[:] + jnp.maximum(acc_p, 0.0)


def _combine_o(aggF, c1, w2, interpret=False):
    whole_w = pl.BlockSpec((2 * D, D), lambda i: (0, 0))
    row = pl.BlockSpec((RB, D), lambda i: (i, 0))
    half = pl.BlockSpec((2, RB, H), lambda i: (0, i, 0))
    return pl.pallas_call(
        _comb_o_body,
        grid=(NB,),
        in_specs=[half, row, whole_w],
        out_specs=row,
        out_shape=jax.ShapeDtypeStruct((F, D), _f32),
        interpret=interpret,
    )(aggF, c1, w2)


def _combine_p(h_p, aggV, c2, w4, interpret=False):
    whole_w = pl.BlockSpec((2 * D, D), lambda i: (0, 0))
    row = pl.BlockSpec((RB, D), lambda i: (i, 0))
    half = pl.BlockSpec((2, RB, H), lambda i: (0, i, 0))
    return pl.pallas_call(
        _comb_p_body,
        grid=(NB,),
        in_specs=[row, half, row, whole_w],
        out_specs=row,
        out_shape=jax.ShapeDtypeStruct((V, D), _f32),
        interpret=interpret,
    )(h_p, aggV, c2, w4)


def kernel(h_p, h_o, edge_index, edge_attr,
           W_v2f_msg, b_v2f_msg, W_v2f_comb, b_v2f_comb,
           W_f2v_msg, b_f2v_msg, W_f2v_comb, b_f2v_comb):
    src = edge_index[0].astype(jnp.int32)
    dst = edge_index[1].astype(jnp.int32)
    b1 = b_v2f_msg.reshape(1, D)
    b2 = b_v2f_comb.reshape(1, D)
    b3 = b_f2v_msg.reshape(1, D)
    b4 = b_f2v_comb.reshape(1, D)

    td1, td2, c1, ts1, ts2, c2 = _precompute(
        h_p, h_o, W_v2f_msg, b1, W_f2v_msg, b3, W_v2f_comb, b2, W_f2v_comb, b4)

    aggF = _edge_sc_phase(
        td1.reshape(2 * F, H), ts1.reshape(2 * V, H), src, dst, True)
    aggV = _edge_sc_phase(
        ts2.reshape(2 * V, H), td2.reshape(2 * F, H), src, dst, False)

    n_h_o = _combine_o(aggF, c1, W_v2f_comb)
    n_h_p = _combine_p(h_p, aggV, c2, W_f2v_comb)
    return (n_h_p, n_h_o)


# speedup vs baseline: 1.0116x; 1.0023x over previous
import functools

import jax
import jax.numpy as jnp
import numpy as np
from jax import lax
from jax.experimental import pallas as pl
from jax.experimental.pallas import tpu as pltpu
from jax.experimental.pallas import tpu_sc as plsc

V = 10000
F = 10000
E = 160000
D = 256
H = 128
NS = 16
CH = 80
EPT = E // NS
BLK = 2000
NBLK = EPT // BLK
CPB = BLK // CH
RB = 1000
NB = F // RB

_f32 = jnp.float32


def _pre1_body(hp, ho, w1, b1, td1, ts1):
    y = jnp.dot(ho[:], w1[0:D], preferred_element_type=_f32) + b1[:]
    td1[0] = y[:, :H]
    td1[1] = y[:, H:]
    y = jnp.dot(hp[:], w1[D:2 * D], preferred_element_type=_f32)
    ts1[0] = y[:, :H]
    ts1[1] = y[:, H:]


def _pre2_body(hp, ho, w3, b3, w2, b2, w4, b4, td2, ts2, c1, c2):
    hp_x = hp[:]
    ho_x = ho[:]
    y = jnp.dot(ho_x, w3[D:2 * D], preferred_element_type=_f32)
    td2[0] = y[:, :H]
    td2[1] = y[:, H:]
    y = jnp.dot(hp_x, w3[0:D], preferred_element_type=_f32) + b3[:]
    ts2[0] = y[:, :H]
    ts2[1] = y[:, H:]
    c1[:] = jnp.dot(ho_x, w2[0:D], preferred_element_type=_f32) + b2[:]
    c2[:] = jnp.dot(hp_x, w4[0:D], preferred_element_type=_f32) + b4[:]


def _precompute1(h_p, h_o, w1, b1, interpret=False):
    whole_w = pl.BlockSpec((2 * D, D), lambda i: (0, 0))
    whole_b = pl.BlockSpec((1, D), lambda i: (0, 0))
    row = pl.BlockSpec((RB, D), lambda i: (i, 0))
    half = pl.BlockSpec((2, RB, H), lambda i: (0, i, 0))
    return pl.pallas_call(
        _pre1_body,
        grid=(NB,),
        in_specs=[row, row, whole_w, whole_b],
        out_specs=[half, half],
        out_shape=[
            jax.ShapeDtypeStruct((2, F, H), _f32),
            jax.ShapeDtypeStruct((2, V, H), _f32),
        ],
        interpret=interpret,
    )(h_p, h_o, w1, b1)


def _precompute2(h_p, h_o, w3, b3, w2, b2, w4, b4, interpret=False):
    whole_w = pl.BlockSpec((2 * D, D), lambda i: (0, 0))
    whole_b = pl.BlockSpec((1, D), lambda i: (0, 0))
    row = pl.BlockSpec((RB, D), lambda i: (i, 0))
    half = pl.BlockSpec((2, RB, H), lambda i: (0, i, 0))
    return pl.pallas_call(
        _pre2_body,
        grid=(NB,),
        in_specs=[row, row, whole_w, whole_b, whole_w, whole_b,
                  whole_w, whole_b],
        out_specs=[half, half, row, row],
        out_shape=[
            jax.ShapeDtypeStruct((2, F, H), _f32),
            jax.ShapeDtypeStruct((2, V, H), _f32),
            jax.ShapeDtypeStruct((F, D), _f32),
            jax.ShapeDtypeStruct((V, D), _f32),
        ],
        interpret=interpret,
    )(h_p, h_o, w3, b3, w2, b2, w4, b4)


ZB = F // CH
ZBT = -(-ZB // NS)


def _sc_phase_body(a_by_dst, tbl_a, tbl_b, src1, dst1, out,
                   ids, idd, igs, igd, scx0, scx1,
                   bufA0, bufB0, bufA1, bufB1,
                   sA0, sB0, sA1, sB1, acc):
    cid = lax.axis_index("c")
    sid = lax.axis_index("s")
    e0 = sid * EPT

    off = cid * F

    @pl.loop(0, CH // 4)
    def _zb(rq):
        r = rq * 4
        for rr in range(4):
            for j in range(H // 16):
                bufA0[r + rr, pl.ds(j * 16, 16)] = jnp.zeros((16,), _f32)

    @pl.loop(0, ZBT)
    def _z(t):
        b = sid + NS * t

        @pl.when(b < ZB)
        def _():
            pltpu.sync_copy(bufA0, acc.at[pl.ds(b * CH, CH)])

    plsc.subcore_barrier()

    ig_a, ig_b = (igd, igs) if a_by_dst else (igs, igd)
    id_main = idd if a_by_dst else ids

    def g_issue(k, ba, bb, sa, sb):
        c0 = k * CH
        pltpu.async_copy(tbl_a.at[ig_a.at[pl.ds(c0, CH)]], ba, sa)
        pltpu.async_copy(tbl_b.at[ig_b.at[pl.ds(c0, CH)]], bb, sb)

    def g_wait(ba, bb, sa, sb):
        pltpu.make_async_copy(
            tbl_a.at[ig_a.at[pl.ds(0, CH)]], ba, sa).wait()
        pltpu.make_async_copy(
            tbl_b.at[ig_b.at[pl.ds(0, CH)]], bb, sb).wait()

    def proc(k, ba, bb, sx):
        for j in range(CH // 16):
            sx[pl.ds(j * 16, 16)] = id_main[pl.ds(k * CH + j * 16, 16)]

        @pl.loop(0, CH // 4)
        def _relu(rq):
            r = rq * 4
            for rr in range(4):
                for j in range(H // 16):
                    s = pl.ds(j * 16, 16)
                    ba[r + rr, s] = jnp.maximum(ba[r + rr, s] + bb[r + rr, s],
                                                0.0)

        pltpu.sync_copy(ba, acc.at[sx], add=True)

    @pl.loop(0, NBLK)
    def _blk(bi):
        base = e0 + bi * BLK
        pltpu.sync_copy(src1.at[pl.ds(base, BLK)], ids)
        pltpu.sync_copy(dst1.at[pl.ds(base, BLK)], idd)

        @pl.loop(0, BLK // 16)
        def _offs(i):
            s = pl.ds(i * 16, 16)
            igs[s] = ids[s] + off
            igd[s] = idd[s] + off

        g_issue(0, bufA0, bufB0, sA0, sB0)

        @pl.loop(0, (CPB - 1) // 2)
        def _pair(kk):
            k0 = kk * 2
            g_issue(k0 + 1, bufA1, bufB1, sA1, sB1)
            g_wait(bufA0, bufB0, sA0, sB0)
            proc(k0, bufA0, bufB0, scx0)
            g_issue(k0 + 2, bufA0, bufB0, sA0, sB0)
            g_wait(bufA1, bufB1, sA1, sB1)
            proc(k0 + 1, bufA1, bufB1, scx1)

        g_wait(bufA0, bufB0, sA0, sB0)
        proc(CPB - 1, bufA0, bufB0, scx0)

    plsc.subcore_barrier()

    @pl.loop(0, ZBT)
    def _w(t):
        b = sid + NS * t

        @pl.when(b < ZB)
        def _():
            pltpu.sync_copy(acc.at[pl.ds(b * CH, CH)],
                            out.at[cid, pl.ds(b * CH, CH)])


def _edge_sc_phase(tbl_a, tbl_b, src1, dst1, a_by_dst):
    mesh = plsc.VectorSubcoreMesh(core_axis_name="c", subcore_axis_name="s")
    fn = pl.kernel(
        functools.partial(_sc_phase_body, a_by_dst),
        out_type=jax.ShapeDtypeStruct((2, F, H), _f32),
        mesh=mesh,
        scratch_types=[
            pltpu.VMEM((BLK,), jnp.int32),
            pltpu.VMEM((BLK,), jnp.int32),
            pltpu.VMEM((BLK,), jnp.int32),
            pltpu.VMEM((BLK,), jnp.int32),
            pltpu.VMEM((CH,), jnp.int32),
            pltpu.VMEM((CH,), jnp.int32),
            pltpu.VMEM((CH, H), _f32),
            pltpu.VMEM((CH, H), _f32),
            pltpu.VMEM((CH, H), _f32),
            pltpu.VMEM((CH, H), _f32),
            pltpu.SemaphoreType.DMA,
            pltpu.SemaphoreType.DMA,
            pltpu.SemaphoreType.DMA,
            pltpu.SemaphoreType.DMA,
            pltpu.VMEM_SHARED((F, H), _f32),
        ],
    )
    return fn(tbl_a, tbl_b, src1, dst1)


def _comb_o_body(af, c1, w2, out_o):
    acc_o = (c1[:]
             + jnp.dot(af[0], w2[D:D + H], preferred_element_type=_f32)
             + jnp.dot(af[1], w2[D + H:2 * D], preferred_element_type=_f32))
    out_o[:] = jnp.maximum(acc_o, 0.0)


def _comb_p_body(hp, av, c2, w4, out_p):
    acc_p = (c2[:]
             + jnp.dot(av[0], w4[D:D + H], preferred_element_type=_f32)
             + jnp.dot(av[1], w4[D + H:2 * D], preferred_element_type=_f32))
    out_p[:] = hp[:] + jnp.maximum(acc_p, 0.0)


def _combine_o(aggF, c1, w2, interpret=False):
    whole_w = pl.BlockSpec((2 * D, D), lambda i: (0, 0))
    row = pl.BlockSpec((RB, D), lambda i: (i, 0))
    half = pl.BlockSpec((2, RB, H), lambda i: (0, i, 0))
    return pl.pallas_call(
        _comb_o_body,
        grid=(NB,),
        in_specs=[half, row, whole_w],
        out_specs=row,
        out_shape=jax.ShapeDtypeStruct((F, D), _f32),
        interpret=interpret,
    )(aggF, c1, w2)


def _combine_p(h_p, aggV, c2, w4, interpret=False):
    whole_w = pl.BlockSpec((2 * D, D), lambda i: (0, 0))
    row = pl.BlockSpec((RB, D), lambda i: (i, 0))
    half = pl.BlockSpec((2, RB, H), lambda i: (0, i, 0))
    return pl.pallas_call(
        _comb_p_body,
        grid=(NB,),
        in_specs=[row, half, row, whole_w],
        out_specs=row,
        out_shape=jax.ShapeDtypeStruct((V, D), _f32),
        interpret=interpret,
    )(h_p, aggV, c2, w4)


def kernel(h_p, h_o, edge_index, edge_attr,
           W_v2f_msg, b_v2f_msg, W_v2f_comb, b_v2f_comb,
           W_f2v_msg, b_f2v_msg, W_f2v_comb, b_f2v_comb):
    src = edge_index[0].astype(jnp.int32)
    dst = edge_index[1].astype(jnp.int32)
    b1 = b_v2f_msg.reshape(1, D)
    b2 = b_v2f_comb.reshape(1, D)
    b3 = b_f2v_msg.reshape(1, D)
    b4 = b_f2v_comb.reshape(1, D)

    td1, ts1 = _precompute1(h_p, h_o, W_v2f_msg, b1)
    aggF = _edge_sc_phase(
        td1.reshape(2 * F, H), ts1.reshape(2 * V, H), src, dst, True)

    td2, ts2, c1, c2 = _precompute2(
        h_p, h_o, W_f2v_msg, b3, W_v2f_comb, b2, W_f2v_comb, b4)
    aggV = _edge_sc_phase(
        ts2.reshape(2 * V, H), td2.reshape(2 * F, H), src, dst, False)

    n_h_o = _combine_o(aggF, c1, W_v2f_comb)
    n_h_p = _combine_p(h_p, aggV, c2, W_f2v_comb)
    return (n_h_p, n_h_o)
